# baseline (device time: 4803825 ns/iter reference)
import jax
import jax.numpy as jnp
from jax import lax
from jax.experimental import pallas as pl
from jax.experimental.pallas import tpu as pltpu

N = 4
M = 4096
D = 4096
CR = 128
NC = M // CR
EPS = 1e-6
F32 = jnp.float32


def kernel(partial, resid, gamma):
    gamma2 = gamma.reshape(1, D)

    def body(x_hbm, resid_hbm, gamma_ref, out_hbm,
             xbuf, residbuf, rrecv, lrecv, rsend, lsend, obuf,
             xsems, ressems, osems,
             rsend_sems, rrecv_sems, lsend_sems, lrecv_sems,
             rcred, lcred):
        xi = lax.axis_index("x")
        yi = lax.axis_index("y")
        zi = lax.axis_index("z")
        right = (xi, yi, jnp.minimum(zi + 1, N - 1))
        left = (xi, yi, jnp.maximum(zi - 1, 0))
        is_z0 = zi == 0
        is_z3 = zi == N - 1
        not_z0 = jnp.logical_not(is_z0)
        not_z3 = jnp.logical_not(is_z3)
        is_mid = jnp.logical_and(not_z0, not_z3)
        is_edge = jnp.logical_or(is_z0, is_z3)
        r_first = zi <= N
        l_first = jnp.logical_not(r_first)

        def AND(*ps):
            out = ps[0]
            for p in ps[1:]:
                out = jnp.logical_and(out, p)
            return out

        bar = pltpu.get_barrier_semaphore()

        @pl.when(not_z0)
        def _():
            pl.semaphore_signal(bar, 1, device_id=left)

        @pl.when(not_z3)
        def _():
            pl.semaphore_signal(bar, 1, device_id=right)

        @pl.when(is_edge)
        def _():
            pl.semaphore_wait(bar, 1)

        @pl.when(is_mid)
        def _():
            pl.semaphore_wait(bar, 2)

        def desc_r(src, s):
            return pltpu.make_async_remote_copy(
                src_ref=src, dst_ref=rrecv.at[s],
                send_sem=rsend_sems.at[s], recv_sem=rrecv_sems.at[s],
                device_id=right)

        def desc_l(src, s):
            return pltpu.make_async_remote_copy(
                src_ref=src, dst_ref=lrecv.at[s],
                send_sem=lsend_sems.at[s], recv_sem=lrecv_sems.at[s],
                device_id=left)

        def copy(src, dst, sem):
            return pltpu.make_async_copy(src, dst, sem)

        def rows(c):
            return pl.ds(c * CR, CR)

        def xload(c, s):
            return copy(x_hbm.at[0, rows(c), :], xbuf.at[s], xsems.at[s])

        def resload(c, s):
            return copy(resid_hbm.at[rows(c), :], residbuf.at[s],
                        ressems.at[s])

        def ostore(c, s):
            return copy(obuf.at[s], out_hbm.at[rows(c), :], osems.at[s])

        xload(0, 0).start()
        resload(0, 0).start()

        def r_relay(act, s4, warm4):
            @pl.when(AND(act, is_z0, warm4))
            def _():
                pl.semaphore_wait(rcred, 1)

            @pl.when(AND(act, is_z0))
            def _():
                desc_r(xbuf.at[s4], s4).start()

            @pl.when(AND(act, not_z0))
            def _():
                desc_r(rsend.at[s4], s4).wait_recv()

            @pl.when(AND(act, is_mid, warm4))
            def _():
                desc_r(rsend.at[s4], s4).wait_send()

            @pl.when(AND(act, is_mid))
            def _():
                rsend[s4, :, :] = rrecv[s4, :, :] + xbuf[s4, :, :]
                pl.semaphore_signal(rcred, 1, device_id=left)

            @pl.when(AND(act, is_mid, warm4))
            def _():
                pl.semaphore_wait(rcred, 1)

            @pl.when(AND(act, is_mid))
            def _():
                desc_r(rsend.at[s4], s4).start()

        def l_relay(act, s4, warm4):
            @pl.when(AND(act, is_z3, warm4))
            def _():
                pl.semaphore_wait(lcred, 1)

            @pl.when(AND(act, is_z3))
            def _():
                desc_l(xbuf.at[s4], s4).start()

            @pl.when(AND(act, not_z3))
            def _():
                desc_l(lsend.at[s4], s4).wait_recv()

            @pl.when(AND(act, is_mid, warm4))
            def _():
                desc_l(lsend.at[s4], s4).wait_send()

            @pl.when(AND(act, is_mid))
            def _():
                lsend[s4, :, :] = lrecv[s4, :, :] + xbuf[s4, :, :]

            @pl.when(AND(act, is_mid, warm4))
            def _():
                pl.semaphore_wait(lcred, 1)

            @pl.when(AND(act, is_mid))
            def _():
                desc_l(lsend.at[s4], s4).start()

        def step(i, _):
            s4 = lax.rem(i, 4)
            warm2 = i >= 2
            warm4 = i >= 4
            relay = i < NC
            o = jnp.maximum(i - 2, 0)
            o4 = lax.rem(o, 4)
            o2 = lax.rem(o, 2)

            @pl.when(relay)
            def _():
                xload(jnp.minimum(i, NC - 1), s4).wait()

            r_relay(AND(relay, r_first), s4, warm4)
            l_relay(AND(relay, r_first), s4, warm4)
            l_relay(AND(relay, l_first), s4, warm4)
            r_relay(AND(relay, l_first), s4, warm4)

            @pl.when(warm2)
            def _():
                resload(o, o4).wait()

            @pl.when(warm4)
            def _():
                ostore(jnp.maximum(i - 4, 0), o2).wait()

            def norm(ybase):
                y = ybase + residbuf[o4, :, :]
                ms = jnp.mean(y * y, axis=-1, keepdims=True)
                obuf[o2, :, :] = (y * lax.rsqrt(ms + EPS)) * gamma_ref[...]

            @pl.when(AND(warm2, is_z0))
            def _():
                norm(xbuf[o4, :, :] + lrecv[o4, :, :])

            @pl.when(AND(warm2, is_mid))
            def _():
                norm(rsend[o4, :, :] + lrecv[o4, :, :])

            @pl.when(AND(warm2, is_z3))
            def _():
                norm(xbuf[o4, :, :] + rrecv[o4, :, :])

            @pl.when(AND(warm2, not_z3))
            def _():
                pl.semaphore_signal(lcred, 1, device_id=right)

            @pl.when(AND(warm2, is_z3))
            def _():
                pl.semaphore_signal(rcred, 1, device_id=left)

            @pl.when(warm2)
            def _():
                ostore(o, o2).start()

            more = i < NC - 1
            ip = jnp.minimum(i + 1, NC - 1)
            ip4 = lax.rem(ip, 4)

            @pl.when(AND(more, is_z0, i >= 3))
            def _():
                desc_r(xbuf.at[ip4], ip4).wait_send()

            @pl.when(AND(more, is_z3, i >= 3))
            def _():
                desc_l(xbuf.at[ip4], ip4).wait_send()

            @pl.when(more)
            def _():
                xload(ip, ip4).start()
                resload(ip, ip4).start()

            return 0

        lax.fori_loop(0, NC + 2, step, 0)

        for c in (NC - 2, NC - 1):
            ostore(c, c % 2).wait()

        @pl.when(is_z0)
        def _():
            for c in range(NC - 4, NC):
                desc_r(xbuf.at[c % 4], c % 4).wait_send()
            pl.semaphore_wait(rcred, 4)

        @pl.when(is_z3)
        def _():
            for c in range(NC - 4, NC):
                desc_l(xbuf.at[c % 4], c % 4).wait_send()
            pl.semaphore_wait(lcred, 4)

        @pl.when(is_mid)
        def _():
            for c in range(NC - 4, NC):
                desc_r(rsend.at[c % 4], c % 4).wait_send()
                desc_l(lsend.at[c % 4], c % 4).wait_send()
            pl.semaphore_wait(rcred, 4)
            pl.semaphore_wait(lcred, 4)

    return pl.pallas_call(
        body,
        out_shape=jax.ShapeDtypeStruct((M, D), F32),
        in_specs=[
            pl.BlockSpec(memory_space=pltpu.MemorySpace.HBM),
            pl.BlockSpec(memory_space=pltpu.MemorySpace.HBM),
            pl.BlockSpec(memory_space=pltpu.MemorySpace.VMEM),
        ],
        out_specs=pl.BlockSpec(memory_space=pltpu.MemorySpace.HBM),
        scratch_shapes=[
            pltpu.VMEM((4, CR, D), F32),
            pltpu.VMEM((4, CR, D), F32),
            pltpu.VMEM((4, CR, D), F32),
            pltpu.VMEM((4, CR, D), F32),
            pltpu.VMEM((4, CR, D), F32),
            pltpu.VMEM((4, CR, D), F32),
            pltpu.VMEM((2, CR, D), F32),
            pltpu.SemaphoreType.DMA((4,)),
            pltpu.SemaphoreType.DMA((4,)),
            pltpu.SemaphoreType.DMA((2,)),
            pltpu.SemaphoreType.DMA((4,)),
            pltpu.SemaphoreType.DMA((4,)),
            pltpu.SemaphoreType.DMA((4,)),
            pltpu.SemaphoreType.DMA((4,)),
            pltpu.SemaphoreType.REGULAR,
            pltpu.SemaphoreType.REGULAR,
        ],
        compiler_params=pltpu.CompilerParams(
            collective_id=0, vmem_limit_bytes=62 * 1024 * 1024),
    )(partial, resid, gamma2)


# device time: 1187388 ns/iter; 4.0457x vs baseline; 4.0457x over previous
import jax
import jax.numpy as jnp
from jax import lax
from jax.experimental import pallas as pl
from jax.experimental.pallas import tpu as pltpu

N = 4
M = 4096
D = 4096
ROUNDS = 2
RH = M // ROUNDS
CH = RH // N
CH2 = CH // 2
TR = 128
EPS = 1e-6
F32 = jnp.float32


def kernel(partial, resid, gamma):
    gamma2 = gamma.reshape(1, D)

    def body(x_hbm, resid_hbm, gamma_ref, out_hbm,
             pbuf, send_buf, rs_recv, ag_recv,
             send_sems, recv_sems, copy_sems, out_sems, credit_sems):
        xi = lax.axis_index("x")
        yi = lax.axis_index("y")
        zi = lax.axis_index("z")
        right = (xi, yi, (zi + 1) % N)
        left = (xi, yi, (zi - 1) % N)

        bar = pltpu.get_barrier_semaphore()
        pl.semaphore_signal(bar, 1, device_id=left)
        pl.semaphore_signal(bar, 1, device_id=right)
        pl.semaphore_wait(bar, 2)

        def copy(src, dst, sem):
            cp = pltpu.make_async_copy(src, dst, sem)
            cp.start()
            return cp

        def tiled(f):
            lax.fori_loop(0, CH2 // TR,
                          lambda t, _: (f(pl.ds(t * TR, TR)), 0)[1], 0)

        DIRS = (
            dict(i=0, tgt=right, csrc=left, off=0,
                 step=lambda s: (zi - s) % N, own=(zi + 1) % N),
            dict(i=1, tgt=left, csrc=right, off=CH2,
                 step=lambda s: (zi + s) % N, own=(zi - 1) % N),
        )

        def one_round(base):
            def rows(d, c):
                return pl.ds(base + c * CH + d["off"], CH2)

            cps = []
            for d in DIRS:
                i = d["i"]
                cps.append(copy(x_hbm.at[0, rows(d, d["step"](0)), :],
                                send_buf.at[i], copy_sems.at[i]))
                cps.append(copy(x_hbm.at[0, rows(d, d["step"](1)), :],
                                pbuf.at[i], copy_sems.at[2 + i]))
            for cp in cps:
                cp.wait()

            pload = [None, None]
            for s in range(N - 1):
                slot = s % 2
                rdmas = []
                for d in DIRS:
                    i = d["i"]
                    rdma = pltpu.make_async_remote_copy(
                        src_ref=send_buf.at[i],
                        dst_ref=rs_recv.at[i, slot],
                        send_sem=send_sems.at[i, s],
                        recv_sem=recv_sems.at[i, s],
                        device_id=d["tgt"],
                    )
                    if s == 2:
                        pl.semaphore_wait(credit_sems.at[i], 1)
                    rdma.start()
                    rdmas.append(rdma)
                for d, rdma in zip(DIRS, rdmas):
                    i = d["i"]
                    rdma.wait()
                    if pload[i] is not None:
                        pload[i].wait()

                    def add(ts, i=i, slot=slot):
                        send_buf[i, ts, :] = (
                            rs_recv[i, slot, ts, :] + pbuf[i, ts, :])
                    tiled(add)
                    if s == 0:
                        pl.semaphore_signal(credit_sems.at[i], 1,
                                            device_id=d["csrc"])
                    if s < 2:
                        pload[i] = copy(
                            x_hbm.at[0, rows(d, d["step"](s + 2)), :],
                            pbuf.at[i], copy_sems.at[2 + i])

            own_cps = []
            for d in DIRS:
                i = d["i"]
                copy(resid_hbm.at[rows(d, d["own"]), :], pbuf.at[i],
                     copy_sems.at[i]).wait()

                def norm(ts, i=i):
                    y = send_buf[i, ts, :] + pbuf[i, ts, :]
                    ms = jnp.mean(y * y, axis=-1, keepdims=True)
                    send_buf[i, ts, :] = (
                        (y * lax.rsqrt(ms + EPS)) * gamma_ref[...])
                tiled(norm)
                own_cps.append(copy(send_buf.at[i],
                                    out_hbm.at[rows(d, d["own"]), :],
                                    copy_sems.at[2 + i]))

            out_cp = [[None, None], [None, None]]
            for s in range(N - 1):
                slot = s % 2
                rdmas = []
                for d in DIRS:
                    i = d["i"]
                    src = send_buf.at[i] if s == 0 else ag_recv.at[i, s - 1]
                    rdma = pltpu.make_async_remote_copy(
                        src_ref=src,
                        dst_ref=ag_recv.at[i, slot],
                        send_sem=send_sems.at[i, 3 + s],
                        recv_sem=recv_sems.at[i, 3 + s],
                        device_id=d["tgt"],
                    )
                    if s == 2:
                        pl.semaphore_wait(credit_sems.at[i], 1)
                    rdma.start()
                    rdmas.append(rdma)
                for d, rdma in zip(DIRS, rdmas):
                    i = d["i"]
                    rdma.wait()
                    if s == 1:
                        out_cp[i][0].wait()
                        pl.semaphore_signal(credit_sems.at[i], 1,
                                            device_id=d["csrc"])
                    org = (zi - s) % N if i == 0 else (zi + s) % N
                    out_cp[i][slot] = copy(
                        ag_recv.at[i, slot], out_hbm.at[rows(d, org), :],
                        out_sems.at[i, slot])
            for cps in out_cp:
                for cp in cps:
                    cp.wait()
            for cp in own_cps:
                cp.wait()

        for r in range(ROUNDS):
            one_round(r * RH)

    return pl.pallas_call(
        body,
        out_shape=jax.ShapeDtypeStruct((M, D), F32),
        in_specs=[
            pl.BlockSpec(memory_space=pltpu.MemorySpace.HBM),
            pl.BlockSpec(memory_space=pltpu.MemorySpace.HBM),
            pl.BlockSpec(memory_space=pltpu.MemorySpace.VMEM),
        ],
        out_specs=pl.BlockSpec(memory_space=pltpu.MemorySpace.HBM),
        scratch_shapes=[
            pltpu.VMEM((2, CH2, D), F32),
            pltpu.VMEM((2, CH2, D), F32),
            pltpu.VMEM((2, 2, CH2, D), F32),
            pltpu.VMEM((2, 2, CH2, D), F32),
            pltpu.SemaphoreType.DMA((2, 6)),
            pltpu.SemaphoreType.DMA((2, 6)),
            pltpu.SemaphoreType.DMA((4,)),
            pltpu.SemaphoreType.DMA((2, 2)),
            pltpu.SemaphoreType.REGULAR((2,)),
        ],
        compiler_params=pltpu.CompilerParams(
            collective_id=0, vmem_limit_bytes=60 * 1024 * 1024),
    )(partial, resid, gamma2)
